# baseline (speedup 1.0000x reference)
import functools

import jax
import jax.numpy as jnp
from jax import lax
from jax.experimental import pallas as pl
from jax.experimental.pallas import tpu as pltpu
from jax.experimental.pallas import tpu_sc as plsc

B = 1024
D = 1024
DFF = 1024
E = 8
K = 2
GH = 512
CH = 512
OUT = 3

BLK = 256
NBLK = 15
NPAD = NBLK * BLK
NW = 32
CHUNK = B // NW


def _gating_body(x_ref, Wg1_ref, bg1_ref, Wg2_ref, bg2_ref,
                 probs_ref, topp_ref, pos_ref, bexp_ref):
    x = x_ref[...]
    h = jnp.maximum(
        jnp.dot(x, Wg1_ref[...], preferred_element_type=jnp.float32)
        + bg1_ref[...],
        0.0,
    )
    logits = (
        jnp.dot(h, Wg2_ref[...], preferred_element_type=jnp.float32)
        + bg2_ref[...]
    )
    m = jnp.max(logits, axis=1, keepdims=True)
    ex = jnp.exp(logits - m)
    probs = ex / jnp.sum(ex, axis=1, keepdims=True)
    probs_ref[...] = probs

    ii = lax.broadcasted_iota(jnp.int32, (B, E), 1)
    m1 = jnp.max(probs, axis=1, keepdims=True)
    i1 = jnp.min(jnp.where(probs == m1, ii, E), axis=1, keepdims=True)
    masked = jnp.where(ii == i1, -1.0, probs)
    m2 = jnp.max(masked, axis=1, keepdims=True)
    i2 = jnp.min(jnp.where(masked == m2, ii, E), axis=1, keepdims=True)
    topp_ref[...] = jnp.concatenate([m1, m2], axis=1)

    oh0 = (ii == i1).astype(jnp.float32)
    oh1 = (ii == i2).astype(jnp.float32)
    tr = lax.broadcasted_iota(jnp.int32, (B, B), 0)
    tc = lax.broadcasted_iota(jnp.int32, (B, B), 1)
    T = (tc < tr).astype(jnp.float32)
    excl0 = jnp.dot(T, oh0, preferred_element_type=jnp.float32)
    excl1 = jnp.dot(T, oh1, preferred_element_type=jnp.float32)
    sum0 = jnp.sum(oh0, axis=0, keepdims=True)
    sum1 = jnp.sum(oh1, axis=0, keepdims=True)
    n = sum0 + sum1
    pc = jnp.ceil(n / BLK) * BLK
    e_lt = (
        lax.broadcasted_iota(jnp.int32, (E, E), 0)
        < lax.broadcasted_iota(jnp.int32, (E, E), 1)
    ).astype(jnp.float32)
    pad_off = jnp.dot(pc, e_lt, preferred_element_type=jnp.float32)

    posm0 = excl0 + pad_off
    posm1 = excl1 + sum0 + pad_off
    pos0 = jnp.sum(jnp.where(ii == i1, posm0, 0.0), axis=1, keepdims=True)
    pos1 = jnp.sum(jnp.where(ii == i2, posm1, 0.0), axis=1, keepdims=True)
    pos_ref[...] = jnp.concatenate([pos0, pos1], axis=1).astype(jnp.int32)

    pad_end = jnp.broadcast_to(pad_off + pc, (16, E))
    gi = (lax.broadcasted_iota(jnp.int32, (16, E), 0) * BLK).astype(
        jnp.float32
    )
    bexp = jnp.sum((gi >= pad_end).astype(jnp.int32), axis=1, keepdims=True)
    bexp_ref[...] = jnp.minimum(bexp, E - 1)


def _ffn_body(bexp_ref, xs_ref, We1_ref, be1_ref, We2_ref, be2_ref, y_ref):
    xb = xs_ref[...].astype(jnp.bfloat16)
    h = jnp.maximum(
        jnp.dot(xb, We1_ref[0], preferred_element_type=jnp.float32)
        + be1_ref[0],
        0.0,
    )
    y_ref[...] = (
        jnp.dot(h.astype(jnp.bfloat16), We2_ref[0],
                preferred_element_type=jnp.float32)
        + be2_ref[0]
    )


def _head_body(c_ref, Wc1_ref, bc1_ref, Wc2_ref, bc2_ref, out_ref):
    h = jnp.maximum(
        jnp.dot(c_ref[...], Wc1_ref[...], preferred_element_type=jnp.float32)
        + bc1_ref[...],
        0.0,
    )
    out_ref[...] = (
        jnp.dot(h, Wc2_ref[...], preferred_element_type=jnp.float32)
        + bc2_ref[...]
    )


@functools.cache
def _get_dispatch():
    mesh = plsc.VectorSubcoreMesh(core_axis_name="c", subcore_axis_name="s")

    H = CHUNK // 2

    @functools.partial(
        pl.kernel,
        out_type=jax.ShapeDtypeStruct((NPAD, D), jnp.float32),
        mesh=mesh,
        scratch_types=[
            pltpu.VMEM((H, D), jnp.float32),
            pltpu.VMEM((H, D), jnp.float32),
            pltpu.VMEM((H,), jnp.int32),
            pltpu.VMEM((H,), jnp.int32),
            pltpu.VMEM((H,), jnp.int32),
            pltpu.VMEM((H,), jnp.int32),
            pltpu.SemaphoreType.DMA,
            pltpu.SemaphoreType.DMA,
            pltpu.SemaphoreType.DMA,
            pltpu.SemaphoreType.DMA,
            pltpu.SemaphoreType.DMA,
            pltpu.SemaphoreType.DMA,
        ],
    )
    def _dispatch(x_hbm, posf_hbm, xs_hbm, xa_v, xb_v,
                  i0a_v, i0b_v, i1a_v, i1b_v,
                  sa, sb, s0, s1, s2, s3):
        wid = lax.axis_index("s") * 2 + lax.axis_index("c")
        base = wid * CHUNK
        lxa = pltpu.async_copy(x_hbm.at[pl.ds(base, H)], xa_v, sa)
        lxb = pltpu.async_copy(x_hbm.at[pl.ds(base + H, H)], xb_v, sb)
        l0a = pltpu.async_copy(posf_hbm.at[pl.ds(base, H)], i0a_v, s0)
        l0b = pltpu.async_copy(posf_hbm.at[pl.ds(base + H, H)], i0b_v, s1)
        l1a = pltpu.async_copy(posf_hbm.at[pl.ds(B + base, H)], i1a_v, s2)
        l1b = pltpu.async_copy(posf_hbm.at[pl.ds(B + base + H, H)],
                               i1b_v, s3)
        lxa.wait()
        l0a.wait()
        l1a.wait()
        ca0 = pltpu.async_copy(xa_v, xs_hbm.at[i0a_v], sa)
        ca1 = pltpu.async_copy(xa_v, xs_hbm.at[i1a_v], s0)
        lxb.wait()
        l0b.wait()
        l1b.wait()
        cb0 = pltpu.async_copy(xb_v, xs_hbm.at[i0b_v], sb)
        cb1 = pltpu.async_copy(xb_v, xs_hbm.at[i1b_v], s1)
        ca0.wait()
        ca1.wait()
        cb0.wait()
        cb1.wait()

    return _dispatch


@functools.cache
def _get_combine():
    mesh = plsc.VectorSubcoreMesh(core_axis_name="c", subcore_axis_name="s")

    H = CHUNK // 2

    @functools.partial(
        pl.kernel,
        out_type=jax.ShapeDtypeStruct((B, D), jnp.float32),
        mesh=mesh,
        scratch_types=[
            pltpu.VMEM((CHUNK, D), jnp.float32),
            pltpu.VMEM((CHUNK, D), jnp.float32),
            pltpu.VMEM((H,), jnp.int32),
            pltpu.VMEM((H,), jnp.int32),
            pltpu.VMEM((H,), jnp.int32),
            pltpu.VMEM((H,), jnp.int32),
            pltpu.VMEM((CHUNK,), jnp.float32),
            pltpu.VMEM((CHUNK,), jnp.float32),
            pltpu.SemaphoreType.DMA,
            pltpu.SemaphoreType.DMA,
            pltpu.SemaphoreType.DMA,
            pltpu.SemaphoreType.DMA,
        ],
    )
    def _combine(y_hbm, posf_hbm, toppf_hbm, out_hbm,
                 y0_v, y1_v, i0a_v, i0b_v, i1a_v, i1b_v, p0_v, p1_v,
                 s0, s1, s2, s3):
        wid = lax.axis_index("s") * 2 + lax.axis_index("c")
        base = wid * CHUNK
        l0 = pltpu.async_copy(posf_hbm.at[pl.ds(base, H)], i0a_v, s0)
        l1 = pltpu.async_copy(posf_hbm.at[pl.ds(base + H, H)], i0b_v, s1)
        l2 = pltpu.async_copy(posf_hbm.at[pl.ds(B + base, H)], i1a_v, s2)
        l3 = pltpu.async_copy(posf_hbm.at[pl.ds(B + base + H, H)],
                              i1b_v, s3)
        pltpu.sync_copy(toppf_hbm.at[pl.ds(base, CHUNK)], p0_v)
        pltpu.sync_copy(toppf_hbm.at[pl.ds(B + base, CHUNK)], p1_v)
        l0.wait()
        l1.wait()
        l2.wait()
        l3.wait()
        ga0 = pltpu.async_copy(y_hbm.at[i0a_v], y0_v.at[pl.ds(0, H)], s0)
        ga1 = pltpu.async_copy(y_hbm.at[i1a_v], y1_v.at[pl.ds(0, H)], s1)
        gb0 = pltpu.async_copy(y_hbm.at[i0b_v], y0_v.at[pl.ds(H, H)], s2)
        gb1 = pltpu.async_copy(y_hbm.at[i1b_v], y1_v.at[pl.ds(H, H)], s3)

        out_cps = []
        for h in range(2):
            if h == 0:
                ga0.wait()
                ga1.wait()
            else:
                gb0.wait()
                gb1.wait()
            pv0 = p0_v[pl.ds(h * H, 16)]
            pv1 = p1_v[pl.ds(h * H, 16)]
            for j16 in range(H):
                j = h * H + j16
                p0s = jnp.broadcast_to(pv0[j16], (16,))
                p1s = jnp.broadcast_to(pv1[j16], (16,))

                def crow(c8, carry, j=j, p0s=p0s, p1s=p1s):
                    off = c8 * 128
                    for c in range(8):
                        sl = pl.ds(off + c * 16, 16)
                        y0_v[j, sl] = (y0_v[j, sl] * p0s
                                       + y1_v[j, sl] * p1s)
                    return carry

                lax.fori_loop(0, D // 128, crow, 0)
            out_cps.append(pltpu.async_copy(
                y0_v.at[pl.ds(h * H, H)],
                out_hbm.at[pl.ds(base + h * H, H)],
                s0 if h == 0 else s1,
            ))
        out_cps[0].wait()
        out_cps[1].wait()

    return _combine


def kernel(x, Wg1, bg1, Wg2, bg2, We1, be1, We2, be2, Wc1, bc1, Wc2, bc2):
    probs, topp, pos, bexp_col = pl.pallas_call(
        _gating_body,
        out_shape=(
            jax.ShapeDtypeStruct((B, E), jnp.float32),
            jax.ShapeDtypeStruct((B, K), jnp.float32),
            jax.ShapeDtypeStruct((B, K), jnp.int32),
            jax.ShapeDtypeStruct((16, 1), jnp.int32),
        ),
    )(x, Wg1, bg1.reshape(1, GH), Wg2, bg2.reshape(1, E))

    posf = pos.T.reshape(K * B)
    toppf = topp.T.reshape(K * B)
    bexp = bexp_col.reshape(16)

    xs = _get_dispatch()(x, posf)

    y = pl.pallas_call(
        _ffn_body,
        grid_spec=pltpu.PrefetchScalarGridSpec(
            num_scalar_prefetch=1,
            grid=(NBLK,),
            in_specs=[
                pl.BlockSpec((BLK, D), lambda g, s: (g, 0)),
                pl.BlockSpec((1, D, DFF), lambda g, s: (s[g], 0, 0)),
                pl.BlockSpec((1, 1, DFF), lambda g, s: (s[g], 0, 0)),
                pl.BlockSpec((1, DFF, D), lambda g, s: (s[g], 0, 0)),
                pl.BlockSpec((1, 1, D), lambda g, s: (s[g], 0, 0)),
            ],
            out_specs=pl.BlockSpec((BLK, D), lambda g, s: (g, 0)),
        ),
        out_shape=jax.ShapeDtypeStruct((NPAD, D), jnp.float32),
    )(bexp, xs, We1.astype(jnp.bfloat16), be1.reshape(E, 1, DFF),
      We2.astype(jnp.bfloat16), be2.reshape(E, 1, D))

    combined = _get_combine()(y, posf, toppf)

    logits = pl.pallas_call(
        _head_body,
        out_shape=jax.ShapeDtypeStruct((B, OUT), jnp.float32),
    )(combined, Wc1, bc1.reshape(1, CH), Wc2, bc2.reshape(1, OUT))

    return (logits, probs)


# speedup vs baseline: 1.2811x; 1.2811x over previous
import functools

import jax
import jax.numpy as jnp
from jax import lax
from jax.experimental import pallas as pl
from jax.experimental.pallas import tpu as pltpu
from jax.experimental.pallas import tpu_sc as plsc

B = 1024
D = 1024
DFF = 1024
E = 8
K = 2
GH = 512
CH = 512
OUT = 3

BLK = 256
NBLK = 15
NPAD = NBLK * BLK
NW = 32
CHUNK = B // NW


def _gating_body(x_ref, Wg1_ref, bg1_ref, Wg2_ref, bg2_ref,
                 probs_ref, topp_ref, pos_ref, bexp_ref):
    x = x_ref[...]
    h = jnp.maximum(
        jnp.dot(x, Wg1_ref[...], preferred_element_type=jnp.float32)
        + bg1_ref[...],
        0.0,
    )
    logits = (
        jnp.dot(h, Wg2_ref[...], preferred_element_type=jnp.float32)
        + bg2_ref[...]
    )
    m = jnp.max(logits, axis=1, keepdims=True)
    ex = jnp.exp(logits - m)
    probs = ex / jnp.sum(ex, axis=1, keepdims=True)
    probs_ref[...] = probs

    ii = lax.broadcasted_iota(jnp.int32, (B, E), 1)
    m1 = jnp.max(probs, axis=1, keepdims=True)
    i1 = jnp.min(jnp.where(probs == m1, ii, E), axis=1, keepdims=True)
    masked = jnp.where(ii == i1, -1.0, probs)
    m2 = jnp.max(masked, axis=1, keepdims=True)
    i2 = jnp.min(jnp.where(masked == m2, ii, E), axis=1, keepdims=True)
    topp_ref[...] = jnp.concatenate([m1, m2], axis=1)

    oh0 = (ii == i1).astype(jnp.float32)
    oh1 = (ii == i2).astype(jnp.float32)
    tr = lax.broadcasted_iota(jnp.int32, (B, B), 0)
    tc = lax.broadcasted_iota(jnp.int32, (B, B), 1)
    T = (tc < tr).astype(jnp.float32)
    excl0 = jnp.dot(T, oh0, preferred_element_type=jnp.float32)
    excl1 = jnp.dot(T, oh1, preferred_element_type=jnp.float32)
    sum0 = jnp.sum(oh0, axis=0, keepdims=True)
    sum1 = jnp.sum(oh1, axis=0, keepdims=True)
    n = sum0 + sum1
    pc = jnp.ceil(n / BLK) * BLK
    e_lt = (
        lax.broadcasted_iota(jnp.int32, (E, E), 0)
        < lax.broadcasted_iota(jnp.int32, (E, E), 1)
    ).astype(jnp.float32)
    pad_off = jnp.dot(pc, e_lt, preferred_element_type=jnp.float32)

    posm0 = excl0 + pad_off
    posm1 = excl1 + sum0 + pad_off
    pos0 = jnp.sum(jnp.where(ii == i1, posm0, 0.0), axis=1, keepdims=True)
    pos1 = jnp.sum(jnp.where(ii == i2, posm1, 0.0), axis=1, keepdims=True)
    pos_ref[...] = jnp.concatenate([pos0, pos1], axis=1).astype(jnp.int32)

    pad_end = jnp.broadcast_to(pad_off + pc, (16, E))
    gi = (lax.broadcasted_iota(jnp.int32, (16, E), 0) * BLK).astype(
        jnp.float32
    )
    bexp = jnp.sum((gi >= pad_end).astype(jnp.int32), axis=1, keepdims=True)
    bexp_ref[...] = jnp.minimum(bexp, E - 1)


def _ffn_body(bexp_ref, xs_ref, We1_ref, be1_ref, We2_ref, be2_ref, y_ref):
    h = jnp.maximum(
        jnp.dot(xs_ref[...], We1_ref[0], preferred_element_type=jnp.float32)
        + be1_ref[0],
        0.0,
    )
    y_ref[...] = (
        jnp.dot(h, We2_ref[0], preferred_element_type=jnp.float32)
        + be2_ref[0]
    )


def _head_body(c_ref, Wc1_ref, bc1_ref, Wc2_ref, bc2_ref, out_ref):
    h = jnp.maximum(
        jnp.dot(c_ref[...], Wc1_ref[...], preferred_element_type=jnp.float32)
        + bc1_ref[...],
        0.0,
    )
    out_ref[...] = (
        jnp.dot(h, Wc2_ref[...], preferred_element_type=jnp.float32)
        + bc2_ref[...]
    )


@functools.cache
def _get_dispatch():
    mesh = plsc.VectorSubcoreMesh(core_axis_name="c", subcore_axis_name="s")

    H = CHUNK // 2

    @functools.partial(
        pl.kernel,
        out_type=jax.ShapeDtypeStruct((NPAD, D), jnp.float32),
        mesh=mesh,
        scratch_types=[
            pltpu.VMEM((H, D), jnp.float32),
            pltpu.VMEM((H, D), jnp.float32),
            pltpu.VMEM((H,), jnp.int32),
            pltpu.VMEM((H,), jnp.int32),
            pltpu.VMEM((H,), jnp.int32),
            pltpu.VMEM((H,), jnp.int32),
            pltpu.SemaphoreType.DMA,
            pltpu.SemaphoreType.DMA,
            pltpu.SemaphoreType.DMA,
            pltpu.SemaphoreType.DMA,
            pltpu.SemaphoreType.DMA,
            pltpu.SemaphoreType.DMA,
        ],
    )
    def _dispatch(x_hbm, posf_hbm, xs_hbm, xa_v, xb_v,
                  i0a_v, i0b_v, i1a_v, i1b_v,
                  sa, sb, s0, s1, s2, s3):
        wid = lax.axis_index("s") * 2 + lax.axis_index("c")
        base = wid * CHUNK
        lxa = pltpu.async_copy(x_hbm.at[pl.ds(base, H)], xa_v, sa)
        lxb = pltpu.async_copy(x_hbm.at[pl.ds(base + H, H)], xb_v, sb)
        l0a = pltpu.async_copy(posf_hbm.at[pl.ds(base, H)], i0a_v, s0)
        l0b = pltpu.async_copy(posf_hbm.at[pl.ds(base + H, H)], i0b_v, s1)
        l1a = pltpu.async_copy(posf_hbm.at[pl.ds(B + base, H)], i1a_v, s2)
        l1b = pltpu.async_copy(posf_hbm.at[pl.ds(B + base + H, H)],
                               i1b_v, s3)
        lxa.wait()
        l0a.wait()
        l1a.wait()
        ca0 = pltpu.async_copy(xa_v, xs_hbm.at[i0a_v], sa)
        ca1 = pltpu.async_copy(xa_v, xs_hbm.at[i1a_v], s0)
        lxb.wait()
        l0b.wait()
        l1b.wait()
        cb0 = pltpu.async_copy(xb_v, xs_hbm.at[i0b_v], sb)
        cb1 = pltpu.async_copy(xb_v, xs_hbm.at[i1b_v], s1)
        ca0.wait()
        ca1.wait()
        cb0.wait()
        cb1.wait()

    return _dispatch


@functools.cache
def _get_combine():
    mesh = plsc.VectorSubcoreMesh(core_axis_name="c", subcore_axis_name="s")

    H = CHUNK // 2

    @functools.partial(
        pl.kernel,
        out_type=jax.ShapeDtypeStruct((B, D), jnp.float32),
        mesh=mesh,
        scratch_types=[
            pltpu.VMEM((CHUNK, D), jnp.float32),
            pltpu.VMEM((CHUNK, D), jnp.float32),
            pltpu.VMEM((H,), jnp.int32),
            pltpu.VMEM((H,), jnp.int32),
            pltpu.VMEM((H,), jnp.int32),
            pltpu.VMEM((H,), jnp.int32),
            pltpu.VMEM((CHUNK,), jnp.float32),
            pltpu.VMEM((CHUNK,), jnp.float32),
            pltpu.SemaphoreType.DMA,
            pltpu.SemaphoreType.DMA,
            pltpu.SemaphoreType.DMA,
            pltpu.SemaphoreType.DMA,
        ],
    )
    def _combine(y_hbm, posf_hbm, toppf_hbm, out_hbm,
                 y0_v, y1_v, i0a_v, i0b_v, i1a_v, i1b_v, p0_v, p1_v,
                 s0, s1, s2, s3):
        wid = lax.axis_index("s") * 2 + lax.axis_index("c")
        base = wid * CHUNK
        l0 = pltpu.async_copy(posf_hbm.at[pl.ds(base, H)], i0a_v, s0)
        l1 = pltpu.async_copy(posf_hbm.at[pl.ds(base + H, H)], i0b_v, s1)
        l2 = pltpu.async_copy(posf_hbm.at[pl.ds(B + base, H)], i1a_v, s2)
        l3 = pltpu.async_copy(posf_hbm.at[pl.ds(B + base + H, H)],
                              i1b_v, s3)
        pltpu.sync_copy(toppf_hbm.at[pl.ds(base, CHUNK)], p0_v)
        pltpu.sync_copy(toppf_hbm.at[pl.ds(B + base, CHUNK)], p1_v)
        l0.wait()
        l1.wait()
        l2.wait()
        l3.wait()
        ga0 = pltpu.async_copy(y_hbm.at[i0a_v], y0_v.at[pl.ds(0, H)], s0)
        ga1 = pltpu.async_copy(y_hbm.at[i1a_v], y1_v.at[pl.ds(0, H)], s1)
        gb0 = pltpu.async_copy(y_hbm.at[i0b_v], y0_v.at[pl.ds(H, H)], s2)
        gb1 = pltpu.async_copy(y_hbm.at[i1b_v], y1_v.at[pl.ds(H, H)], s3)

        out_cps = []
        for h in range(2):
            if h == 0:
                ga0.wait()
                ga1.wait()
            else:
                gb0.wait()
                gb1.wait()
            pv0 = p0_v[pl.ds(h * H, 16)]
            pv1 = p1_v[pl.ds(h * H, 16)]
            for j16 in range(H):
                j = h * H + j16
                p0s = jnp.broadcast_to(pv0[j16], (16,))
                p1s = jnp.broadcast_to(pv1[j16], (16,))

                def crow(c8, carry, j=j, p0s=p0s, p1s=p1s):
                    off = c8 * 128
                    for c in range(8):
                        sl = pl.ds(off + c * 16, 16)
                        y0_v[j, sl] = (y0_v[j, sl] * p0s
                                       + y1_v[j, sl] * p1s)
                    return carry

                lax.fori_loop(0, D // 128, crow, 0)
            out_cps.append(pltpu.async_copy(
                y0_v.at[pl.ds(h * H, H)],
                out_hbm.at[pl.ds(base + h * H, H)],
                s0 if h == 0 else s1,
            ))
        out_cps[0].wait()
        out_cps[1].wait()

    return _combine


def kernel(x, Wg1, bg1, Wg2, bg2, We1, be1, We2, be2, Wc1, bc1, Wc2, bc2):
    probs, topp, pos, bexp_col = pl.pallas_call(
        _gating_body,
        out_shape=(
            jax.ShapeDtypeStruct((B, E), jnp.float32),
            jax.ShapeDtypeStruct((B, K), jnp.float32),
            jax.ShapeDtypeStruct((B, K), jnp.int32),
            jax.ShapeDtypeStruct((16, 1), jnp.int32),
        ),
    )(x, Wg1, bg1.reshape(1, GH), Wg2, bg2.reshape(1, E))

    posf = pos.T.reshape(K * B)
    toppf = topp.T.reshape(K * B)
    bexp = bexp_col.reshape(16)

    xs = _get_dispatch()(x, posf)

    y = pl.pallas_call(
        _ffn_body,
        grid_spec=pltpu.PrefetchScalarGridSpec(
            num_scalar_prefetch=1,
            grid=(NBLK,),
            in_specs=[
                pl.BlockSpec((BLK, D), lambda g, s: (g, 0)),
                pl.BlockSpec((1, D, DFF), lambda g, s: (s[g], 0, 0)),
                pl.BlockSpec((1, 1, DFF), lambda g, s: (s[g], 0, 0)),
                pl.BlockSpec((1, DFF, D), lambda g, s: (s[g], 0, 0)),
                pl.BlockSpec((1, 1, D), lambda g, s: (s[g], 0, 0)),
            ],
            out_specs=pl.BlockSpec((BLK, D), lambda g, s: (g, 0)),
        ),
        out_shape=jax.ShapeDtypeStruct((NPAD, D), jnp.float32),
    )(bexp, xs, We1, be1.reshape(E, 1, DFF), We2, be2.reshape(E, 1, D))

    combined = _get_combine()(y, posf, toppf)

    logits = pl.pallas_call(
        _head_body,
        out_shape=jax.ShapeDtypeStruct((B, OUT), jnp.float32),
    )(combined, Wc1, bc1.reshape(1, CH), Wc2, bc2.reshape(1, OUT))

    return (logits, probs)


# speedup vs baseline: 1.4503x; 1.1321x over previous
import functools

import jax
import jax.numpy as jnp
from jax import lax
from jax.experimental import pallas as pl
from jax.experimental.pallas import tpu as pltpu
from jax.experimental.pallas import tpu_sc as plsc

B = 1024
D = 1024
DFF = 1024
E = 8
K = 2
GH = 512
CH = 512
OUT = 3

NW = 32
CHUNK = B // NW


def _gating_body(x_ref, Wg1_ref, bg1_ref, Wg2_ref, bg2_ref,
                 probs_ref, probsT_ref):
    h = jnp.maximum(
        jnp.dot(x_ref[...], Wg1_ref[...], preferred_element_type=jnp.float32)
        + bg1_ref[...],
        0.0,
    )
    logits = (
        jnp.dot(h, Wg2_ref[...], preferred_element_type=jnp.float32)
        + bg2_ref[...]
    )
    m = jnp.max(logits, axis=1, keepdims=True)
    ex = jnp.exp(logits - m)
    probs = ex / jnp.sum(ex, axis=1, keepdims=True)
    probs_ref[...] = probs
    probsT_ref[...] = jnp.transpose(probs)


@functools.cache
def _get_route():
    mesh = plsc.VectorSubcoreMesh(core_axis_name="c", subcore_axis_name="s")

    @functools.partial(
        pl.kernel,
        out_type=jax.ShapeDtypeStruct((E * B,), jnp.float32),
        mesh=mesh,
        scratch_types=[
            pltpu.VMEM((E, CHUNK), jnp.float32),
            pltpu.VMEM((E, CHUNK), jnp.float32),
            pltpu.SemaphoreType.DMA,
            pltpu.SemaphoreType.DMA,
        ],
    )
    def _route(probsT_hbm, wt_hbm, pv_v, wt_v, sem, osem):
        wid = lax.axis_index("s") * 2 + lax.axis_index("c")
        base = wid * CHUNK
        cps = [
            pltpu.async_copy(probsT_hbm.at[pl.ds(e * B + base, CHUNK)],
                             pv_v.at[e], sem)
            for e in range(E)
        ]
        for cp in cps:
            cp.wait()
        for h in range(CHUNK // 16):
            sl = pl.ds(h * 16, 16)
            pe = [pv_v[e, sl] for e in range(E)]
            m1 = pe[0]
            for e in range(1, E):
                m1 = jnp.maximum(m1, pe[e])
            i1 = jnp.full((16,), E, jnp.int32)
            for e in range(E - 1, -1, -1):
                i1 = jnp.where(pe[e] == m1, e, i1)
            pm = [jnp.where(i1 == e, -1.0, pe[e]) for e in range(E)]
            m2 = pm[0]
            for e in range(1, E):
                m2 = jnp.maximum(m2, pm[e])
            i2 = jnp.full((16,), E, jnp.int32)
            for e in range(E - 1, -1, -1):
                i2 = jnp.where(pm[e] == m2, e, i2)
            for e in range(E):
                wt_v[e, sl] = (jnp.where(i1 == e, m1, 0.0)
                               + jnp.where(i2 == e, m2, 0.0))
        ocps = [
            pltpu.async_copy(wt_v.at[e],
                             wt_hbm.at[pl.ds(e * B + base, CHUNK)], osem)
            for e in range(E)
        ]
        for cp in ocps:
            cp.wait()

    return _route


def _moe_body(x_ref, We1_ref, be1_ref, We2_ref, be2_ref, wT_ref,
              Wc1_ref, bc1_ref, Wc2_ref, bc2_ref, out_ref, acc_ref):
    e = pl.program_id(0)
    h = jnp.maximum(
        jnp.dot(x_ref[...], We1_ref[0], preferred_element_type=jnp.float32)
        + be1_ref[0],
        0.0,
    )
    y = (
        jnp.dot(h, We2_ref[0], preferred_element_type=jnp.float32)
        + be2_ref[0]
    )
    ee = lax.broadcasted_iota(jnp.int32, (E, B), 0)
    row = jnp.sum(jnp.where(ee == e, wT_ref[...], 0.0), axis=0,
                  keepdims=True)
    contrib = jnp.transpose(row) * y

    @pl.when(e == 0)
    def _():
        acc_ref[...] = contrib

    @pl.when(e != 0)
    def _():
        acc_ref[...] += contrib

    @pl.when(e == E - 1)
    def _():
        hc = jnp.maximum(
            jnp.dot(acc_ref[...], Wc1_ref[...],
                    preferred_element_type=jnp.float32)
            + bc1_ref[...],
            0.0,
        )
        out_ref[...] = (
            jnp.dot(hc, Wc2_ref[...], preferred_element_type=jnp.float32)
            + bc2_ref[...]
        )


def kernel(x, Wg1, bg1, Wg2, bg2, We1, be1, We2, be2, Wc1, bc1, Wc2, bc2):
    probs, probsT = pl.pallas_call(
        _gating_body,
        out_shape=(
            jax.ShapeDtypeStruct((B, E), jnp.float32),
            jax.ShapeDtypeStruct((E, B), jnp.float32),
        ),
    )(x, Wg1, bg1.reshape(1, GH), Wg2, bg2.reshape(1, E))

    wT = _get_route()(probsT.reshape(E * B)).reshape(E, B)

    logits = pl.pallas_call(
        _moe_body,
        grid=(E,),
        in_specs=[
            pl.BlockSpec((B, D), lambda e: (0, 0)),
            pl.BlockSpec((1, D, DFF), lambda e: (e, 0, 0)),
            pl.BlockSpec((1, 1, DFF), lambda e: (e, 0, 0)),
            pl.BlockSpec((1, DFF, D), lambda e: (e, 0, 0)),
            pl.BlockSpec((1, 1, D), lambda e: (e, 0, 0)),
            pl.BlockSpec((E, B), lambda e: (0, 0)),
            pl.BlockSpec((D, CH), lambda e: (0, 0)),
            pl.BlockSpec((1, CH), lambda e: (0, 0)),
            pl.BlockSpec((CH, OUT), lambda e: (0, 0)),
            pl.BlockSpec((1, OUT), lambda e: (0, 0)),
        ],
        out_specs=pl.BlockSpec((B, OUT), lambda e: (0, 0)),
        out_shape=jax.ShapeDtypeStruct((B, OUT), jnp.float32),
        scratch_shapes=[pltpu.VMEM((B, D), jnp.float32)],
    )(x, We1, be1.reshape(E, 1, DFF), We2, be2.reshape(E, 1, D), wT,
      Wc1, bc1.reshape(1, CH), Wc2, bc2.reshape(1, OUT))

    return (logits, probs)
